# baseline (speedup 1.0000x reference)
import functools

import jax
import jax.numpy as jnp
from jax import lax
from jax.experimental import pallas as pl
from jax.experimental.pallas import tpu as pltpu
from jax.experimental.pallas import tpu_sc as plsc

N = 10000
E = 320000
D1 = 128
D2 = 64
NPROTO = 16
MLPH = 8

NC = 2
NS = 16
NW = NC * NS
EPT = E // NW
C = 80
NCH = EPT // C
_WB_FOR_D = {1: 400, D1: 80, D2: 80}


def _make_sc_deg():
  mesh = plsc.VectorSubcoreMesh(core_axis_name="c", subcore_axis_name="s")
  scratch = [
      pltpu.VMEM((EPT,), jnp.int32),
      pltpu.VMEM((N,), jnp.float32),
  ]

  @functools.partial(
      pl.kernel, mesh=mesh,
      out_type=jax.ShapeDtypeStruct((NW, 1, N), jnp.float32),
      scratch_types=scratch,
      compiler_params=pltpu.CompilerParams(needs_layout_passes=False))
  def k(dst_hbm, z_hbm, out_hbm, dst_v, hist_v):
    cid = lax.axis_index("c")
    sid = lax.axis_index("s")
    wid = cid * NS + sid
    pltpu.sync_copy(z_hbm, hist_v)
    pltpu.sync_copy(dst_hbm.at[pl.ds(wid * EPT, EPT)], dst_v)
    ones = jnp.ones((16,), jnp.float32)
    UNROLL = 5

    def body(i, carry):
      for u in range(UNROLL):
        idx = dst_v[pl.ds((i * UNROLL + u) * 16, 16)]
        plsc.addupdate_scatter(hist_v, [idx], ones)
      return carry

    lax.fori_loop(0, EPT // 16 // UNROLL, body, 0)
    pltpu.sync_copy(hist_v, out_hbm.at[wid, 0])

  return k


def _make_sc_agg(D, tc_tiling=True):
  WB = _WB_FOR_D[D]
  NRCH = N // WB
  mesh = plsc.VectorSubcoreMesh(core_axis_name="c", subcore_axis_name="s")
  scratch = [
      pltpu.VMEM((EPT,), jnp.int32),
      pltpu.VMEM((EPT,), jnp.int32),
      pltpu.VMEM((C, D), jnp.float32),
      pltpu.VMEM((C, D), jnp.float32),
      pltpu.VMEM((C, D), jnp.float32),
      pltpu.VMEM_SHARED((N, D), jnp.float32),
      [pltpu.SemaphoreType.DMA] * 3,
      [pltpu.SemaphoreType.DMA] * 3,
      pltpu.SemaphoreType.DMA,
  ]
  NZ = (NRCH + NS - 1) // NS

  @functools.partial(
      pl.kernel, mesh=mesh,
      out_type=jax.ShapeDtypeStruct((NC, N, D), jnp.float32),
      scratch_types=scratch,
      compiler_params=pltpu.CompilerParams(use_tc_tiling_on_sc=tc_tiling))
  def k(src_hbm, dst_hbm, g_hbm, z_hbm, out_hbm,
        si_v, di_v, r0, r1, r2, accum, gs, ts, sz):
    s0, s1 = gs[0], gs[1]
    cid = lax.axis_index("c")
    sid = lax.axis_index("s")
    wid = cid * NS + sid
    ci = pltpu.async_copy(src_hbm.at[pl.ds(wid * EPT, EPT)], si_v, s0)
    cd = pltpu.async_copy(dst_hbm.at[pl.ds(wid * EPT, EPT)], di_v, s1)
    zb = r1.at[pl.ds(0, WB)]
    pltpu.sync_copy(z_hbm, zb)
    for j in range(NZ):
      ch = j * NS + sid

      @pl.when(ch < NRCH)
      def _():
        pltpu.async_copy(zb, accum.at[pl.ds(ch * WB, WB)], sz)
    for j in range(NZ):
      ch = j * NS + sid

      @pl.when(ch < NRCH)
      def _():
        pltpu.make_async_copy(zb, accum.at[pl.ds(ch * WB, WB)], sz).wait()
    ci.wait()
    cd.wait()
    plsc.subcore_barrier()

    rb = (r0, r1, r2)

    def gslice(i):
      return g_hbm.at[si_v.at[pl.ds(i * C, C)]]

    def sref(i):
      return accum.at[di_v.at[pl.ds(i * C, C)]]

    pltpu.async_copy(gslice(0), rb[0], gs[0])
    pltpu.async_copy(gslice(1), rb[1], gs[1])

    def step(i, off, first=False, last=False):
      b = off % 3
      pltpu.make_async_copy(gslice(i), rb[b], gs[b]).wait()
      pltpu.async_copy(rb[b], sref(i), ts[b], add=True)
      if not first:
        pb = (off + 2) % 3
        pltpu.make_async_copy(rb[pb], sref(i - 1), ts[pb]).wait()
        if not last:
          pltpu.async_copy(gslice(i + 2), rb[pb], gs[pb])

    step(0, 0, first=True)
    pltpu.async_copy(gslice(2), rb[2], gs[2])
    step(1, 1)
    step(2, 2)

    def body(k, carry):
      i = 3 * k
      step(i, 0)
      step(i + 1, 1)
      step(i + 2, 2)
      return carry

    assert NCH % 3 == 2, "pipeline assumes NCH = 3m+2"
    lax.fori_loop(1, (NCH - 2) // 3, body, 0)
    step(NCH - 2, 0, last=True)
    step(NCH - 1, 1, last=True)
    pltpu.make_async_copy(rb[1], sref(NCH - 1), ts[1]).wait()
    plsc.subcore_barrier()

    def wb_buf(j):
      r = r0 if j % 2 == 0 else r1
      return r.at[pl.ds(0, WB)], (s0 if j % 2 == 0 else s1)

    for j in range(NZ):
      ch = j * NS + sid
      buf, sem = wb_buf(j)
      if j >= 2:
        pch = (j - 2) * NS + sid
        pltpu.make_async_copy(buf, out_hbm.at[cid, pl.ds(pch * WB, WB)],
                              sem).wait()

      def do_wb(ch=ch, buf=buf, sem=sem):
        pltpu.sync_copy(accum.at[pl.ds(ch * WB, WB)], buf)
        pltpu.async_copy(buf, out_hbm.at[cid, pl.ds(ch * WB, WB)], sem)

      if j < NZ - 1:
        do_wb()
      else:
        pl.when(ch < NRCH)(do_wb)
    ch = (NZ - 2) * NS + sid
    buf, sem = wb_buf(NZ - 2)
    pltpu.make_async_copy(buf, out_hbm.at[cid, pl.ds(ch * WB, WB)], sem).wait()
    ch2 = (NZ - 1) * NS + sid
    buf2, sem2 = wb_buf(NZ - 1)

    @pl.when(ch2 < NRCH)
    def _():
      pltpu.make_async_copy(buf2, out_hbm.at[cid, pl.ds(ch2 * WB, WB)],
                            sem2).wait()

  return k


_deg_sc = _make_sc_deg()
_agg_sc = _make_sc_agg(D1)
_agg2_sc = _make_sc_agg(D2, tc_tiling=False)


def _tc1(dp_ref, x_ref, w1_ref, g_ref, dinv_ref):
  dp2 = jnp.squeeze(dp_ref[...], axis=1)
  ones_w = jnp.ones((NW, 1), jnp.float32)
  deg = lax.dot_general(dp2, ones_w, (((0,), (0,)), ((), ())),
                        preferred_element_type=jnp.float32) + 1.0
  dinv = lax.rsqrt(deg)
  dinv_ref[...] = dinv
  h = jnp.dot(x_ref[...], w1_ref[...], preferred_element_type=jnp.float32)
  g_ref[...] = h * dinv


def _tc2(q_ref, g1_ref, dinv_ref, b1_ref, w2_ref, g2_ref):
  dinv = dinv_ref[...]
  pre = (q_ref[0] + q_ref[1] + g1_ref[...]) * dinv + b1_ref[...]
  out1 = jnp.maximum(pre, 0.0)
  h2 = jnp.dot(out1, w2_ref[...], preferred_element_type=jnp.float32)
  g2_ref[...] = h2 * dinv


def _tc3(r_ref, g2_ref, dinv_ref, b2_ref, p_ref, mw0_ref, mb0_ref,
         mw1_ref, mb1_ref, s_ref):
  dinv = dinv_ref[...]
  pre = (r_ref[0] + r_ref[1] + g2_ref[...]) * dinv + b2_ref[...]
  emb = jnp.maximum(pre, 0.0)
  p = p_ref[...]
  pn = jnp.sum(p * p, axis=1)
  en = jnp.sum(emb * emb, axis=1, keepdims=True)
  cross = lax.dot_general(emb, p, (((1,), (1,)), ((), ())),
                          preferred_element_type=jnp.float32)
  dist = en + pn[None, :] - 2.0 * cross
  sim = jnp.log(dist + 1.0) - jnp.log(dist + 1e-4)
  z = jax.nn.gelu(jnp.dot(sim, mw0_ref[...],
                          preferred_element_type=jnp.float32) + mb0_ref[...])
  z = jnp.dot(z, mw1_ref[...], preferred_element_type=jnp.float32) + mb1_ref[...]
  s_ref[...] = jax.nn.sigmoid(z)


def kernel(x, edge_index, y, W1, b1, W2, b2, prototypes, mW0, mb0, mW1, mb1):
  src = edge_index[0].astype(jnp.int32)
  dst = edge_index[1].astype(jnp.int32)
  zrow = jnp.zeros((N,), jnp.float32)
  zerosD1 = jnp.zeros((_WB_FOR_D[D1], D1), jnp.float32)
  zerosD2 = jnp.zeros((_WB_FOR_D[D2], D2), jnp.float32)

  dp = _deg_sc(dst, zrow)

  g1, dinv = pl.pallas_call(
      _tc1,
      out_shape=[jax.ShapeDtypeStruct((N, D1), jnp.float32),
                 jax.ShapeDtypeStruct((N, 1), jnp.float32)],
  )(dp, x, W1)

  q = _agg_sc(src, dst, g1, zerosD1)

  g2 = pl.pallas_call(
      _tc2,
      out_shape=jax.ShapeDtypeStruct((N, D2), jnp.float32),
  )(q, g1, dinv, b1, W2)

  r = _agg2_sc(src, dst, g2, zerosD2)

  scores = pl.pallas_call(
      _tc3,
      out_shape=jax.ShapeDtypeStruct((N, 1), jnp.float32),
  )(r, g2, dinv, b2, prototypes, mW0, mb0, mW1, mb1)

  return (jnp.squeeze(scores, axis=-1), y.astype(jnp.float32))


# speedup vs baseline: 1.0016x; 1.0016x over previous
import functools

import jax
import jax.numpy as jnp
from jax import lax
from jax.experimental import pallas as pl
from jax.experimental.pallas import tpu as pltpu
from jax.experimental.pallas import tpu_sc as plsc

N = 10000
E = 320000
D1 = 128
D2 = 64
NPROTO = 16
MLPH = 8

NC = 2
NS = 16
NW = NC * NS
EPT = E // NW
C = 80
NCH = EPT // C
_WB_FOR_D = {1: 400, D1: 80, D2: 80}


def _make_sc_deg():
  mesh = plsc.VectorSubcoreMesh(core_axis_name="c", subcore_axis_name="s")
  scratch = [
      pltpu.VMEM((EPT,), jnp.int32),
      pltpu.VMEM((N,), jnp.float32),
  ]

  @functools.partial(
      pl.kernel, mesh=mesh,
      out_type=jax.ShapeDtypeStruct((NW, 1, N), jnp.float32),
      scratch_types=scratch,
      compiler_params=pltpu.CompilerParams(needs_layout_passes=False,
                                          skip_device_barrier=True))
  def k(dst_hbm, z_hbm, out_hbm, dst_v, hist_v):
    cid = lax.axis_index("c")
    sid = lax.axis_index("s")
    wid = cid * NS + sid
    pltpu.sync_copy(z_hbm, hist_v)
    pltpu.sync_copy(dst_hbm.at[pl.ds(wid * EPT, EPT)], dst_v)
    ones = jnp.ones((16,), jnp.float32)
    UNROLL = 5

    def body(i, carry):
      for u in range(UNROLL):
        idx = dst_v[pl.ds((i * UNROLL + u) * 16, 16)]
        plsc.addupdate_scatter(hist_v, [idx], ones)
      return carry

    lax.fori_loop(0, EPT // 16 // UNROLL, body, 0)
    pltpu.sync_copy(hist_v, out_hbm.at[wid, 0])

  return k


def _make_sc_agg(D, tc_tiling=True):
  WB = _WB_FOR_D[D]
  NRCH = N // WB
  mesh = plsc.VectorSubcoreMesh(core_axis_name="c", subcore_axis_name="s")
  scratch = [
      pltpu.VMEM((EPT,), jnp.int32),
      pltpu.VMEM((EPT,), jnp.int32),
      pltpu.VMEM((C, D), jnp.float32),
      pltpu.VMEM((C, D), jnp.float32),
      pltpu.VMEM((C, D), jnp.float32),
      pltpu.VMEM_SHARED((N, D), jnp.float32),
      [pltpu.SemaphoreType.DMA] * 3,
      [pltpu.SemaphoreType.DMA] * 3,
      pltpu.SemaphoreType.DMA,
  ]
  NZ = (NRCH + NS - 1) // NS

  @functools.partial(
      pl.kernel, mesh=mesh,
      out_type=jax.ShapeDtypeStruct((NC, N, D), jnp.float32),
      scratch_types=scratch,
      compiler_params=pltpu.CompilerParams(use_tc_tiling_on_sc=tc_tiling,
                                          skip_device_barrier=True))
  def k(src_hbm, dst_hbm, g_hbm, z_hbm, out_hbm,
        si_v, di_v, r0, r1, r2, accum, gs, ts, sz):
    s0, s1 = gs[0], gs[1]
    cid = lax.axis_index("c")
    sid = lax.axis_index("s")
    wid = cid * NS + sid
    ci = pltpu.async_copy(src_hbm.at[pl.ds(wid * EPT, EPT)], si_v, s0)
    cd = pltpu.async_copy(dst_hbm.at[pl.ds(wid * EPT, EPT)], di_v, s1)
    zb = r1.at[pl.ds(0, WB)]
    pltpu.sync_copy(z_hbm, zb)
    for j in range(NZ):
      ch = j * NS + sid

      @pl.when(ch < NRCH)
      def _():
        pltpu.async_copy(zb, accum.at[pl.ds(ch * WB, WB)], sz)
    for j in range(NZ):
      ch = j * NS + sid

      @pl.when(ch < NRCH)
      def _():
        pltpu.make_async_copy(zb, accum.at[pl.ds(ch * WB, WB)], sz).wait()
    ci.wait()
    cd.wait()
    plsc.subcore_barrier()

    rb = (r0, r1, r2)

    def gslice(i):
      return g_hbm.at[si_v.at[pl.ds(i * C, C)]]

    def sref(i):
      return accum.at[di_v.at[pl.ds(i * C, C)]]

    pltpu.async_copy(gslice(0), rb[0], gs[0])
    pltpu.async_copy(gslice(1), rb[1], gs[1])

    def step(i, off, first=False, last=False):
      b = off % 3
      pltpu.make_async_copy(gslice(i), rb[b], gs[b]).wait()
      pltpu.async_copy(rb[b], sref(i), ts[b], add=True)
      if not first:
        pb = (off + 2) % 3
        pltpu.make_async_copy(rb[pb], sref(i - 1), ts[pb]).wait()
        if not last:
          pltpu.async_copy(gslice(i + 2), rb[pb], gs[pb])

    step(0, 0, first=True)
    pltpu.async_copy(gslice(2), rb[2], gs[2])
    step(1, 1)
    step(2, 2)

    def body(k, carry):
      i = 3 * k
      step(i, 0)
      step(i + 1, 1)
      step(i + 2, 2)
      return carry

    assert NCH % 3 == 2, "pipeline assumes NCH = 3m+2"
    lax.fori_loop(1, (NCH - 2) // 3, body, 0)
    step(NCH - 2, 0, last=True)
    step(NCH - 1, 1, last=True)
    pltpu.make_async_copy(rb[1], sref(NCH - 1), ts[1]).wait()
    plsc.subcore_barrier()

    def wb_buf(j):
      r = r0 if j % 2 == 0 else r1
      return r.at[pl.ds(0, WB)], (s0 if j % 2 == 0 else s1)

    for j in range(NZ):
      ch = j * NS + sid
      buf, sem = wb_buf(j)
      if j >= 2:
        pch = (j - 2) * NS + sid
        pltpu.make_async_copy(buf, out_hbm.at[cid, pl.ds(pch * WB, WB)],
                              sem).wait()

      def do_wb(ch=ch, buf=buf, sem=sem):
        pltpu.sync_copy(accum.at[pl.ds(ch * WB, WB)], buf)
        pltpu.async_copy(buf, out_hbm.at[cid, pl.ds(ch * WB, WB)], sem)

      if j < NZ - 1:
        do_wb()
      else:
        pl.when(ch < NRCH)(do_wb)
    ch = (NZ - 2) * NS + sid
    buf, sem = wb_buf(NZ - 2)
    pltpu.make_async_copy(buf, out_hbm.at[cid, pl.ds(ch * WB, WB)], sem).wait()
    ch2 = (NZ - 1) * NS + sid
    buf2, sem2 = wb_buf(NZ - 1)

    @pl.when(ch2 < NRCH)
    def _():
      pltpu.make_async_copy(buf2, out_hbm.at[cid, pl.ds(ch2 * WB, WB)],
                            sem2).wait()

  return k


_deg_sc = _make_sc_deg()
_agg_sc = _make_sc_agg(D1)
_agg2_sc = _make_sc_agg(D2, tc_tiling=False)


def _tc1(dp_ref, x_ref, w1_ref, g_ref, dinv_ref):
  dp2 = jnp.squeeze(dp_ref[...], axis=1)
  ones_w = jnp.ones((NW, 1), jnp.float32)
  deg = lax.dot_general(dp2, ones_w, (((0,), (0,)), ((), ())),
                        preferred_element_type=jnp.float32) + 1.0
  dinv = lax.rsqrt(deg)
  dinv_ref[...] = dinv
  h = jnp.dot(x_ref[...], w1_ref[...], preferred_element_type=jnp.float32)
  g_ref[...] = h * dinv


def _tc2(q_ref, g1_ref, dinv_ref, b1_ref, w2_ref, g2_ref):
  dinv = dinv_ref[...]
  pre = (q_ref[0] + q_ref[1] + g1_ref[...]) * dinv + b1_ref[...]
  out1 = jnp.maximum(pre, 0.0)
  h2 = jnp.dot(out1, w2_ref[...], preferred_element_type=jnp.float32)
  g2_ref[...] = h2 * dinv


def _tc3(r_ref, g2_ref, dinv_ref, b2_ref, p_ref, mw0_ref, mb0_ref,
         mw1_ref, mb1_ref, s_ref):
  dinv = dinv_ref[...]
  pre = (r_ref[0] + r_ref[1] + g2_ref[...]) * dinv + b2_ref[...]
  emb = jnp.maximum(pre, 0.0)
  p = p_ref[...]
  pn = jnp.sum(p * p, axis=1)
  en = jnp.sum(emb * emb, axis=1, keepdims=True)
  cross = lax.dot_general(emb, p, (((1,), (1,)), ((), ())),
                          preferred_element_type=jnp.float32)
  dist = en + pn[None, :] - 2.0 * cross
  sim = jnp.log(dist + 1.0) - jnp.log(dist + 1e-4)
  z = jax.nn.gelu(jnp.dot(sim, mw0_ref[...],
                          preferred_element_type=jnp.float32) + mb0_ref[...])
  z = jnp.dot(z, mw1_ref[...], preferred_element_type=jnp.float32) + mb1_ref[...]
  s_ref[...] = jax.nn.sigmoid(z)


def kernel(x, edge_index, y, W1, b1, W2, b2, prototypes, mW0, mb0, mW1, mb1):
  src = edge_index[0].astype(jnp.int32)
  dst = edge_index[1].astype(jnp.int32)
  zrow = jnp.zeros((N,), jnp.float32)
  zerosD1 = jnp.zeros((_WB_FOR_D[D1], D1), jnp.float32)
  zerosD2 = jnp.zeros((_WB_FOR_D[D2], D2), jnp.float32)

  dp = _deg_sc(dst, zrow)

  g1, dinv = pl.pallas_call(
      _tc1,
      out_shape=[jax.ShapeDtypeStruct((N, D1), jnp.float32),
                 jax.ShapeDtypeStruct((N, 1), jnp.float32)],
  )(dp, x, W1)

  q = _agg_sc(src, dst, g1, zerosD1)

  g2 = pl.pallas_call(
      _tc2,
      out_shape=jax.ShapeDtypeStruct((N, D2), jnp.float32),
  )(q, g1, dinv, b1, W2)

  r = _agg2_sc(src, dst, g2, zerosD2)

  scores = pl.pallas_call(
      _tc3,
      out_shape=jax.ShapeDtypeStruct((N, 1), jnp.float32),
  )(r, g2, dinv, b2, prototypes, mW0, mb0, mW1, mb1)

  return (jnp.squeeze(scores, axis=-1), y.astype(jnp.float32))
